# baseline (speedup 1.0000x reference)
import functools

import jax
import jax.numpy as jnp
from jax import lax
from jax.experimental import pallas as pl
from jax.experimental.pallas import tpu as pltpu
from jax.experimental.pallas import tpu_sc as plsc

N = 10000
F_IN = 128
H = 128
OUT = 8
E = 320000
AE_H = 512
NUM_GENES = 5000

RB = 2000
NRB = N // RB
KB = 3200
NKB = (N * OUT) // KB


def _leaky(v):
    return jnp.where(v >= 0, v, 0.01 * v)


def _mm(a, w):
    return lax.dot_general(a, w, (((1,), (1,)), ((), ())),
                           preferred_element_type=jnp.float32)


def _pre_body(x_ref, w_ref, b_ref, out_ref):
    out_ref[...] = lax.dot_general(
        x_ref[...], w_ref[...], (((0,), (1,)), ((), ())),
        preferred_element_type=jnp.float32) + b_ref[...]


def _gin_body(h_ref, msg_ref, w1_ref, b1_ref, w2_ref, b2_ref,
              out_ref, cs_ref, cs2_ref):
    i = pl.program_id(0)
    z = h_ref[...] + msg_ref[0] + msg_ref[1]
    z = _leaky(_mm(z, w1_ref[...]) + b1_ref[...])
    hn = _mm(z, w2_ref[...]) + b2_ref[...]
    out_ref[...] = hn
    s = jnp.sum(hn, axis=0, keepdims=True)
    s2 = jnp.sum(hn * hn, axis=0, keepdims=True)

    @pl.when(i == 0)
    def _():
        cs_ref[...] = s
        cs2_ref[...] = s2

    @pl.when(i > 0)
    def _():
        cs_ref[...] += s
        cs2_ref[...] += s2


def _bn_body(h_ref, cs_ref, cs2_ref, g_ref, b_ref, out_ref):
    m = cs_ref[...] / N
    v = cs2_ref[...] / N - m * m
    out_ref[...] = g_ref[...] * (h_ref[...] - m) * lax.rsqrt(v + 1e-5) \
        + b_ref[...]


def _post_body(h_ref, w1_ref, b1_ref, w2_ref, b2_ref, out_ref):
    z = _leaky(_mm(h_ref[...], w1_ref[...]) + b1_ref[...])
    out_ref[...] = _mm(z, w2_ref[...]) + b2_ref[...]


def _enc0_body(g_ref, w_ref, b_ref, out_ref):
    k = pl.program_id(0)
    part = _mm(g_ref[...], w_ref[...])
    tot = jnp.where(k == 0, b_ref[...], out_ref[...]) + part
    out_ref[...] = jnp.where(k == NKB - 1, jnp.maximum(tot, 0.0), tot)


def _tail_body(e_ref, w1_ref, b1_ref, w2_ref, b2_ref,
               d0_ref, db0_ref, d1_ref, db1_ref, d2_ref, db2_ref, out_ref):
    e = jnp.maximum(_mm(e_ref[...], w1_ref[...]) + b1_ref[...], 0.0)
    e = _mm(e, w2_ref[...]) + b2_ref[...]
    d = jnp.maximum(_mm(e, d0_ref[...]) + db0_ref[...], 0.0)
    d = jnp.maximum(_mm(d, d1_ref[...]) + db1_ref[...], 0.0)
    out_ref[...] = _mm(d, d2_ref[...]) + db2_ref[...]


def _full(shape):
    return pl.BlockSpec(shape, lambda *_: tuple(0 for _ in shape))


def _pre(x, w, b):
    return pl.pallas_call(
        _pre_body,
        in_specs=[_full((F_IN, N)), _full((H, F_IN)), _full((1, H))],
        out_specs=_full((N, H)),
        out_shape=jax.ShapeDtypeStruct((N, H), jnp.float32),
    )(x, w, b)


def _gin(h, msg, w1, b1, w2, b2):
    return pl.pallas_call(
        _gin_body,
        grid=(NRB,),
        in_specs=[pl.BlockSpec((RB, H), lambda i: (i, 0)),
                  pl.BlockSpec((2, RB, H), lambda i: (0, i, 0)),
                  _full((H, H)), _full((1, H)), _full((H, H)), _full((1, H))],
        out_specs=[pl.BlockSpec((RB, H), lambda i: (i, 0)),
                   pl.BlockSpec((1, H), lambda i: (0, 0)),
                   pl.BlockSpec((1, H), lambda i: (0, 0))],
        out_shape=[jax.ShapeDtypeStruct((N, H), jnp.float32),
                   jax.ShapeDtypeStruct((1, H), jnp.float32),
                   jax.ShapeDtypeStruct((1, H), jnp.float32)],
    )(h, msg, w1, b1, w2, b2)


def _bn(h, cs, cs2, g, b):
    return pl.pallas_call(
        _bn_body,
        grid=(NRB,),
        in_specs=[pl.BlockSpec((RB, H), lambda i: (i, 0)),
                  _full((1, H)), _full((1, H)), _full((1, H)), _full((1, H))],
        out_specs=pl.BlockSpec((RB, H), lambda i: (i, 0)),
        out_shape=jax.ShapeDtypeStruct((N, H), jnp.float32),
    )(h, cs, cs2, g, b)


def _post(h, w1, b1, w2, b2):
    return pl.pallas_call(
        _post_body,
        grid=(NRB,),
        in_specs=[pl.BlockSpec((RB, H), lambda i: (i, 0)),
                  _full((H, H)), _full((1, H)), _full((OUT, H)),
                  _full((1, OUT))],
        out_specs=pl.BlockSpec((RB, OUT), lambda i: (i, 0)),
        out_shape=jax.ShapeDtypeStruct((N, OUT), jnp.float32),
    )(h, w1, b1, w2, b2)


def _enc0(g, w, b):
    return pl.pallas_call(
        _enc0_body,
        grid=(NKB,),
        in_specs=[pl.BlockSpec((1, KB), lambda k: (0, k)),
                  pl.BlockSpec((AE_H, KB), lambda k: (0, k)),
                  _full((1, AE_H))],
        out_specs=pl.BlockSpec((1, AE_H), lambda k: (0, 0)),
        out_shape=jax.ShapeDtypeStruct((1, AE_H), jnp.float32),
    )(g, w, b)


def _tail(e, p):
    args = (e, p['enc_W1'], p['enc_b1'][None, :], p['enc_W2'],
            p['enc_b2'][None, :], p['dec_W0'], p['dec_b0'][None, :],
            p['dec_W1'], p['dec_b1'][None, :], p['dec_W2'],
            p['dec_b2'][None, :])
    return pl.pallas_call(
        _tail_body,
        in_specs=[_full(a.shape) for a in args],
        out_specs=_full((1, NUM_GENES)),
        out_shape=jax.ShapeDtypeStruct((1, NUM_GENES), jnp.float32),
    )(*args)


EB = 128
NSUB = 16
NWORK = 2 * NSUB
BLK_W = 80
NBLKP = NWORK * BLK_W
EPAD = NBLKP * EB - E
NCHUNK = N // EB
NREM = N - NCHUNK * EB
NBUF = 2
HALF = 40
NPH0 = 1
NPH1 = 3

_sc_mesh = plsc.VectorSubcoreMesh(core_axis_name="c", subcore_axis_name="s")


@functools.partial(
    pl.kernel, mesh=_sc_mesh,
    out_type=jax.ShapeDtypeStruct((2, N, H), jnp.float32),
    scratch_types=[
        pltpu.VMEM_SHARED((N + EB, H), jnp.float32),
        pltpu.VMEM((HALF, EB), jnp.int32),
        pltpu.VMEM((HALF, EB), jnp.int32),
        pltpu.VMEM((NBUF, EB, H), jnp.float32),
        pltpu.SemaphoreType.DMA,
        pltpu.SemaphoreType.DMA,
    ],
)
def _segsum_sc(h_hbm, src_hbm, dst_hbm, zeros_hbm, out_hbm,
               acc, src_v, dst_v, rows4, *sems):
    rows_v = rows4.at[0]
    c = lax.axis_index("c")
    s = lax.axis_index("s")
    wid = c * NSUB + s

    pltpu.sync_copy(zeros_hbm, rows_v)
    for tt in range(5):
        t = s + NSUB * tt
        off = pl.multiple_of(t * EB, EB)
        if tt < 4:
            pltpu.sync_copy(rows_v, acc.at[pl.ds(off, EB)])
        else:
            @pl.when(t < NCHUNK)
            def _():
                pltpu.sync_copy(rows_v, acc.at[pl.ds(off, EB)])

    @pl.when(s == NSUB - 1)
    def _():
        pltpu.sync_copy(rows_v.at[pl.ds(0, NREM)],
                        acc.at[pl.ds(NCHUNK * EB, NREM)])
        pltpu.sync_copy(rows_v, acc.at[pl.ds(N, EB)])
    plsc.subcore_barrier()

    nph = jnp.where(c == 0, NPH0, NPH1)
    wbase = jnp.where(c == 0, s * (NPH0 * HALF),
                      NPH0 * HALF * NSUB + s * (NPH1 * HALF))

    def phase(ph, pcarry):
        pb = pl.multiple_of(wbase + ph * HALF, 8)
        pltpu.sync_copy(src_hbm.at[pl.ds(pb, HALF)], src_v)
        pltpu.sync_copy(dst_hbm.at[pl.ds(pb, HALF)], dst_v)
        for b in range(NBUF):
            pltpu.async_copy(h_hbm.at[src_v.at[b]], rows4.at[b], sems[b])

        def body(g, carry):
            for b in range(NBUF):
                blk = g * NBUF + b
                pltpu.make_async_copy(h_hbm.at[src_v.at[blk]], rows4.at[b],
                                      sems[b]).wait()
                pltpu.sync_copy(rows4.at[b], acc.at[dst_v.at[blk]], add=True)
                nxt = blk + NBUF

                @pl.when(nxt < HALF)
                def _():
                    pltpu.async_copy(h_hbm.at[src_v.at[nxt]], rows4.at[b],
                                     sems[b])
            return carry

        lax.fori_loop(0, HALF // NBUF, body, 0)
        return pcarry

    lax.fori_loop(0, nph, phase, 0)
    plsc.subcore_barrier()

    for tt in range(5):
        t = s + NSUB * tt
        off = pl.multiple_of(t * EB, EB)
        if tt < 4:
            pltpu.sync_copy(acc.at[pl.ds(off, EB)], rows_v)
            pltpu.sync_copy(rows_v, out_hbm.at[c, pl.ds(off, EB)])
        else:
            @pl.when(t < NCHUNK)
            def _():
                pltpu.sync_copy(acc.at[pl.ds(off, EB)], rows_v)
                pltpu.sync_copy(rows_v, out_hbm.at[c, pl.ds(off, EB)])

    @pl.when(s == NSUB - 1)
    def _():
        pltpu.sync_copy(acc.at[pl.ds(NCHUNK * EB, NREM)],
                        rows_v.at[pl.ds(0, NREM)])
        pltpu.sync_copy(rows_v.at[pl.ds(0, NREM)],
                        out_hbm.at[c, pl.ds(NCHUNK * EB, NREM)])


def _segsum(h, src2d, dst2d, zeros):
    return _segsum_sc(h, src2d, dst2d, zeros)


def kernel(x, edge_index, params):
    p = params
    pad_src = jnp.zeros((EPAD,), jnp.int32)
    pad_dst = jnp.full((EPAD,), N, jnp.int32)
    src2d = jnp.concatenate([edge_index[0], pad_src]).reshape(NBLKP, EB)
    dst2d = jnp.concatenate([edge_index[1], pad_dst]).reshape(NBLKP, EB)
    zeros = jnp.zeros((EB, H), jnp.float32)
    h = _pre(x, p['W_pre'], p['b_pre'][None, :])
    for i in range(3):
        msg = _segsum(h, src2d, dst2d, zeros)
        h, cs, cs2 = _gin(h, msg, p['c%d_W1' % i], p['c%d_b1' % i][None, :],
                          p['c%d_W2' % i], p['c%d_b2' % i][None, :])
        if i < 2:
            h = _bn(h, cs, cs2, p['bn%d_g' % i][None, :],
                    p['bn%d_b' % i][None, :])
    h8 = _post(h, p['post_W1'], p['post_b1'][None, :],
               p['post_W2'], p['post_b2'][None, :])
    g = h8.reshape(1, N * OUT)
    e0 = _enc0(g, p['enc_W0'], p['enc_b0'][None, :])
    return _tail(e0, p)


# speedup vs baseline: 1.0424x; 1.0424x over previous
import functools

import jax
import jax.numpy as jnp
from jax import lax
from jax.experimental import pallas as pl
from jax.experimental.pallas import tpu as pltpu
from jax.experimental.pallas import tpu_sc as plsc

N = 10000
F_IN = 128
H = 128
OUT = 8
E = 320000
AE_H = 512
NUM_GENES = 5000

RB = 2000
NRB = N // RB
KB = 3200
NKB = (N * OUT) // KB


def _leaky(v):
    return jnp.where(v >= 0, v, 0.01 * v)


def _mm(a, w):
    return lax.dot_general(a, w, (((1,), (1,)), ((), ())),
                           preferred_element_type=jnp.float32)


def _pre_body(x_ref, w_ref, b_ref, out_ref):
    out_ref[...] = lax.dot_general(
        x_ref[...], w_ref[...], (((0,), (1,)), ((), ())),
        preferred_element_type=jnp.float32) + b_ref[...]


def _gin_body(h_ref, msg_ref, w1_ref, b1_ref, w2_ref, b2_ref,
              out_ref, cs_ref, cs2_ref):
    i = pl.program_id(0)
    z = h_ref[...] + msg_ref[0] + msg_ref[1]
    z = _leaky(_mm(z, w1_ref[...]) + b1_ref[...])
    hn = _mm(z, w2_ref[...]) + b2_ref[...]
    out_ref[...] = hn
    s = jnp.sum(hn, axis=0, keepdims=True)
    s2 = jnp.sum(hn * hn, axis=0, keepdims=True)

    @pl.when(i == 0)
    def _():
        cs_ref[...] = s
        cs2_ref[...] = s2

    @pl.when(i > 0)
    def _():
        cs_ref[...] += s
        cs2_ref[...] += s2


def _bn_body(h_ref, cs_ref, cs2_ref, g_ref, b_ref, out_ref):
    m = cs_ref[...] / N
    v = cs2_ref[...] / N - m * m
    out_ref[...] = g_ref[...] * (h_ref[...] - m) * lax.rsqrt(v + 1e-5) \
        + b_ref[...]


def _post_body(h_ref, w1_ref, b1_ref, w2_ref, b2_ref, out_ref):
    z = _leaky(_mm(h_ref[...], w1_ref[...]) + b1_ref[...])
    out_ref[...] = _mm(z, w2_ref[...]) + b2_ref[...]


def _enc0_body(g_ref, w_ref, b_ref, out_ref):
    k = pl.program_id(0)
    part = _mm(g_ref[...], w_ref[...])
    tot = jnp.where(k == 0, b_ref[...], out_ref[...]) + part
    out_ref[...] = jnp.where(k == NKB - 1, jnp.maximum(tot, 0.0), tot)


def _tail_body(e_ref, w1_ref, b1_ref, w2_ref, b2_ref,
               d0_ref, db0_ref, d1_ref, db1_ref, d2_ref, db2_ref, out_ref):
    e = jnp.maximum(_mm(e_ref[...], w1_ref[...]) + b1_ref[...], 0.0)
    e = _mm(e, w2_ref[...]) + b2_ref[...]
    d = jnp.maximum(_mm(e, d0_ref[...]) + db0_ref[...], 0.0)
    d = jnp.maximum(_mm(d, d1_ref[...]) + db1_ref[...], 0.0)
    out_ref[...] = _mm(d, d2_ref[...]) + db2_ref[...]


def _full(shape):
    return pl.BlockSpec(shape, lambda *_: tuple(0 for _ in shape))


def _pre(x, w, b):
    return pl.pallas_call(
        _pre_body,
        in_specs=[_full((F_IN, N)), _full((H, F_IN)), _full((1, H))],
        out_specs=_full((N, H)),
        out_shape=jax.ShapeDtypeStruct((N, H), jnp.float32),
    )(x, w, b)


def _gin(h, msg, w1, b1, w2, b2):
    return pl.pallas_call(
        _gin_body,
        grid=(NRB,),
        in_specs=[pl.BlockSpec((RB, H), lambda i: (i, 0)),
                  pl.BlockSpec((2, RB, H), lambda i: (0, i, 0)),
                  _full((H, H)), _full((1, H)), _full((H, H)), _full((1, H))],
        out_specs=[pl.BlockSpec((RB, H), lambda i: (i, 0)),
                   pl.BlockSpec((1, H), lambda i: (0, 0)),
                   pl.BlockSpec((1, H), lambda i: (0, 0))],
        out_shape=[jax.ShapeDtypeStruct((N, H), jnp.float32),
                   jax.ShapeDtypeStruct((1, H), jnp.float32),
                   jax.ShapeDtypeStruct((1, H), jnp.float32)],
    )(h, msg, w1, b1, w2, b2)


def _bn(h, cs, cs2, g, b):
    return pl.pallas_call(
        _bn_body,
        grid=(NRB,),
        in_specs=[pl.BlockSpec((RB, H), lambda i: (i, 0)),
                  _full((1, H)), _full((1, H)), _full((1, H)), _full((1, H))],
        out_specs=pl.BlockSpec((RB, H), lambda i: (i, 0)),
        out_shape=jax.ShapeDtypeStruct((N, H), jnp.float32),
    )(h, cs, cs2, g, b)


def _post(h, w1, b1, w2, b2):
    return pl.pallas_call(
        _post_body,
        grid=(NRB,),
        in_specs=[pl.BlockSpec((RB, H), lambda i: (i, 0)),
                  _full((H, H)), _full((1, H)), _full((OUT, H)),
                  _full((1, OUT))],
        out_specs=pl.BlockSpec((RB, OUT), lambda i: (i, 0)),
        out_shape=jax.ShapeDtypeStruct((N, OUT), jnp.float32),
    )(h, w1, b1, w2, b2)


def _enc0(g, w, b):
    return pl.pallas_call(
        _enc0_body,
        grid=(NKB,),
        in_specs=[pl.BlockSpec((1, KB), lambda k: (0, k)),
                  pl.BlockSpec((AE_H, KB), lambda k: (0, k)),
                  _full((1, AE_H))],
        out_specs=pl.BlockSpec((1, AE_H), lambda k: (0, 0)),
        out_shape=jax.ShapeDtypeStruct((1, AE_H), jnp.float32),
    )(g, w, b)


def _tail(e, p):
    args = (e, p['enc_W1'], p['enc_b1'][None, :], p['enc_W2'],
            p['enc_b2'][None, :], p['dec_W0'], p['dec_b0'][None, :],
            p['dec_W1'], p['dec_b1'][None, :], p['dec_W2'],
            p['dec_b2'][None, :])
    return pl.pallas_call(
        _tail_body,
        in_specs=[_full(a.shape) for a in args],
        out_specs=_full((1, NUM_GENES)),
        out_shape=jax.ShapeDtypeStruct((1, NUM_GENES), jnp.float32),
    )(*args)


EB = 128
NSUB = 16
NWORK = 2 * NSUB
BLK_W = 80
NBLKP = NWORK * BLK_W
EPAD = NBLKP * EB - E
NCHUNK = N // EB
NREM = N - NCHUNK * EB
NBUF = 2
HALF = 40
NPH = BLK_W // HALF


@functools.cache
def _segsum_sc_build():
    mesh = plsc.VectorSubcoreMesh(core_axis_name="c", subcore_axis_name="s")
    return functools.partial(
        pl.kernel, mesh=mesh,
        out_type=jax.ShapeDtypeStruct((2, N, H), jnp.float32),
        scratch_types=[
            pltpu.VMEM_SHARED((N + EB, H), jnp.float32),
            pltpu.VMEM((HALF, EB), jnp.int32),
            pltpu.VMEM((HALF, EB), jnp.int32),
            pltpu.VMEM((NBUF, EB, H), jnp.float32),
            pltpu.SemaphoreType.DMA,
            pltpu.SemaphoreType.DMA,
            pltpu.SemaphoreType.DMA,
            pltpu.SemaphoreType.DMA,
        ],
    )(_segsum_sc_body)


def _segsum_sc_body(h_hbm, src_hbm, dst_hbm, zeros_hbm, out_hbm,
                    acc, src_v, dst_v, rows4, *sems):
    c = lax.axis_index("c")
    s = lax.axis_index("s")
    wid = c * NSUB + s
    rows_v = rows4.at[0]

    pltpu.sync_copy(zeros_hbm, rows_v)
    for tt in range(5):
        t = s + NSUB * tt
        off = pl.multiple_of(t * EB, EB)
        if tt < 4:
            pltpu.sync_copy(rows_v, acc.at[pl.ds(off, EB)])
        else:
            @pl.when(t < NCHUNK)
            def _():
                pltpu.sync_copy(rows_v, acc.at[pl.ds(off, EB)])

    @pl.when(s == NSUB - 1)
    def _():
        pltpu.sync_copy(rows_v.at[pl.ds(0, NREM)],
                        acc.at[pl.ds(NCHUNK * EB, NREM)])
        pltpu.sync_copy(rows_v, acc.at[pl.ds(N, EB)])
    plsc.subcore_barrier()

    for ph in range(NPH):
        pb = pl.multiple_of(wid * BLK_W + ph * HALF, 8)
        pltpu.sync_copy(src_hbm.at[pl.ds(pb, HALF)], src_v)
        pltpu.sync_copy(dst_hbm.at[pl.ds(pb, HALF)], dst_v)
        EH = EB // 2
        for b in range(NBUF):
            for u in range(2):
                pltpu.async_copy(h_hbm.at[src_v.at[b, pl.ds(u * EH, EH)]],
                                 rows4.at[b].at[pl.ds(u * EH, EH)],
                                 sems[2 * b + u])

        def body(g, carry):
            for b in range(NBUF):
                blk = g * NBUF + b
                for u in range(2):
                    pltpu.make_async_copy(
                        h_hbm.at[src_v.at[blk, pl.ds(u * EH, EH)]],
                        rows4.at[b].at[pl.ds(u * EH, EH)],
                        sems[2 * b + u]).wait()
                pltpu.sync_copy(rows4.at[b], acc.at[dst_v.at[blk]], add=True)
                nxt = blk + NBUF

                @pl.when(nxt < HALF)
                def _():
                    for u in range(2):
                        pltpu.async_copy(
                            h_hbm.at[src_v.at[nxt, pl.ds(u * EH, EH)]],
                            rows4.at[b].at[pl.ds(u * EH, EH)],
                            sems[2 * b + u])
            return carry

        lax.fori_loop(0, HALF // NBUF, body, 0)
    plsc.subcore_barrier()

    for tt in range(5):
        t = s + NSUB * tt
        off = pl.multiple_of(t * EB, EB)
        if tt < 4:
            pltpu.sync_copy(acc.at[pl.ds(off, EB)], rows_v)
            pltpu.sync_copy(rows_v, out_hbm.at[c, pl.ds(off, EB)])
        else:
            @pl.when(t < NCHUNK)
            def _():
                pltpu.sync_copy(acc.at[pl.ds(off, EB)], rows_v)
                pltpu.sync_copy(rows_v, out_hbm.at[c, pl.ds(off, EB)])

    @pl.when(s == NSUB - 1)
    def _():
        pltpu.sync_copy(acc.at[pl.ds(NCHUNK * EB, NREM)],
                        rows_v.at[pl.ds(0, NREM)])
        pltpu.sync_copy(rows_v.at[pl.ds(0, NREM)],
                        out_hbm.at[c, pl.ds(NCHUNK * EB, NREM)])


def _segsum(h, src2d, dst2d, zeros):
    return _segsum_sc_build()(h, src2d, dst2d, zeros)


def kernel(x, edge_index, params):
    p = params
    pad_src = jnp.zeros((EPAD,), jnp.int32)
    pad_dst = jnp.full((EPAD,), N, jnp.int32)
    src2d = jnp.concatenate([edge_index[0], pad_src]).reshape(NBLKP, EB)
    dst2d = jnp.concatenate([edge_index[1], pad_dst]).reshape(NBLKP, EB)
    zeros = jnp.zeros((EB, H), jnp.float32)
    h = _pre(x, p['W_pre'], p['b_pre'][None, :])
    for i in range(3):
        msg = _segsum(h, src2d, dst2d, zeros)
        h, cs, cs2 = _gin(h, msg, p['c%d_W1' % i], p['c%d_b1' % i][None, :],
                          p['c%d_W2' % i], p['c%d_b2' % i][None, :])
        if i < 2:
            h = _bn(h, cs, cs2, p['bn%d_g' % i][None, :],
                    p['bn%d_b' % i][None, :])
    h8 = _post(h, p['post_W1'], p['post_b1'][None, :],
               p['post_W2'], p['post_b2'][None, :])
    g = h8.reshape(1, N * OUT)
    e0 = _enc0(g, p['enc_W0'], p['enc_b0'][None, :])
    return _tail(e0, p)


# speedup vs baseline: 1.0452x; 1.0027x over previous
import functools

import jax
import jax.numpy as jnp
from jax import lax
from jax.experimental import pallas as pl
from jax.experimental.pallas import tpu as pltpu
from jax.experimental.pallas import tpu_sc as plsc

N = 10000
F_IN = 128
H = 128
OUT = 8
E = 320000
AE_H = 512
NUM_GENES = 5000

RB = 2000
NRB = N // RB
KB = 3200
NKB = (N * OUT) // KB


def _leaky(v):
    return jnp.where(v >= 0, v, 0.01 * v)


def _mm(a, w):
    return lax.dot_general(a, w, (((1,), (1,)), ((), ())),
                           preferred_element_type=jnp.float32)


def _pre_body(x_ref, w_ref, b_ref, out_ref):
    out_ref[...] = lax.dot_general(
        x_ref[...], w_ref[...], (((0,), (1,)), ((), ())),
        preferred_element_type=jnp.float32) + b_ref[...]


def _gin_body(h_ref, msg_ref, w1_ref, b1_ref, w2_ref, b2_ref,
              out_ref, cs_ref, cs2_ref):
    i = pl.program_id(0)
    z = h_ref[...] + msg_ref[0] + msg_ref[1]
    z = _leaky(_mm(z, w1_ref[...]) + b1_ref[...])
    hn = _mm(z, w2_ref[...]) + b2_ref[...]
    out_ref[...] = hn
    s = jnp.sum(hn, axis=0, keepdims=True)
    s2 = jnp.sum(hn * hn, axis=0, keepdims=True)

    @pl.when(i == 0)
    def _():
        cs_ref[...] = s
        cs2_ref[...] = s2

    @pl.when(i > 0)
    def _():
        cs_ref[...] += s
        cs2_ref[...] += s2


def _bn_body(h_ref, cs_ref, cs2_ref, g_ref, b_ref, out_ref):
    m = cs_ref[...] / N
    v = cs2_ref[...] / N - m * m
    out_ref[...] = g_ref[...] * (h_ref[...] - m) * lax.rsqrt(v + 1e-5) \
        + b_ref[...]


def _post_body(h_ref, w1_ref, b1_ref, w2_ref, b2_ref, out_ref):
    z = _leaky(_mm(h_ref[...], w1_ref[...]) + b1_ref[...])
    out_ref[...] = _mm(z, w2_ref[...]) + b2_ref[...]


def _enc0_body(g_ref, w_ref, b_ref, out_ref):
    k = pl.program_id(0)
    part = _mm(g_ref[...], w_ref[...])
    tot = jnp.where(k == 0, b_ref[...], out_ref[...]) + part
    out_ref[...] = jnp.where(k == NKB - 1, jnp.maximum(tot, 0.0), tot)


def _tail_body(e_ref, w1_ref, b1_ref, w2_ref, b2_ref,
               d0_ref, db0_ref, d1_ref, db1_ref, d2_ref, db2_ref, out_ref):
    e = jnp.maximum(_mm(e_ref[...], w1_ref[...]) + b1_ref[...], 0.0)
    e = _mm(e, w2_ref[...]) + b2_ref[...]
    d = jnp.maximum(_mm(e, d0_ref[...]) + db0_ref[...], 0.0)
    d = jnp.maximum(_mm(d, d1_ref[...]) + db1_ref[...], 0.0)
    out_ref[...] = _mm(d, d2_ref[...]) + db2_ref[...]


def _full(shape):
    return pl.BlockSpec(shape, lambda *_: tuple(0 for _ in shape))


def _pre(x, w, b):
    return pl.pallas_call(
        _pre_body,
        in_specs=[_full((F_IN, N)), _full((H, F_IN)), _full((1, H))],
        out_specs=_full((N, H)),
        out_shape=jax.ShapeDtypeStruct((N, H), jnp.float32),
    )(x, w, b)


def _gin(h, msg, w1, b1, w2, b2):
    return pl.pallas_call(
        _gin_body,
        grid=(NRB,),
        in_specs=[pl.BlockSpec((RB, H), lambda i: (i, 0)),
                  pl.BlockSpec((2, RB, H), lambda i: (0, i, 0)),
                  _full((H, H)), _full((1, H)), _full((H, H)), _full((1, H))],
        out_specs=[pl.BlockSpec((RB, H), lambda i: (i, 0)),
                   pl.BlockSpec((1, H), lambda i: (0, 0)),
                   pl.BlockSpec((1, H), lambda i: (0, 0))],
        out_shape=[jax.ShapeDtypeStruct((N, H), jnp.float32),
                   jax.ShapeDtypeStruct((1, H), jnp.float32),
                   jax.ShapeDtypeStruct((1, H), jnp.float32)],
    )(h, msg, w1, b1, w2, b2)


def _bn(h, cs, cs2, g, b):
    return pl.pallas_call(
        _bn_body,
        grid=(NRB,),
        in_specs=[pl.BlockSpec((RB, H), lambda i: (i, 0)),
                  _full((1, H)), _full((1, H)), _full((1, H)), _full((1, H))],
        out_specs=pl.BlockSpec((RB, H), lambda i: (i, 0)),
        out_shape=jax.ShapeDtypeStruct((N, H), jnp.float32),
    )(h, cs, cs2, g, b)


def _post(h, w1, b1, w2, b2):
    return pl.pallas_call(
        _post_body,
        grid=(NRB,),
        in_specs=[pl.BlockSpec((RB, H), lambda i: (i, 0)),
                  _full((H, H)), _full((1, H)), _full((OUT, H)),
                  _full((1, OUT))],
        out_specs=pl.BlockSpec((RB, OUT), lambda i: (i, 0)),
        out_shape=jax.ShapeDtypeStruct((N, OUT), jnp.float32),
    )(h, w1, b1, w2, b2)


def _enc0(g, w, b):
    return pl.pallas_call(
        _enc0_body,
        grid=(NKB,),
        in_specs=[pl.BlockSpec((1, KB), lambda k: (0, k)),
                  pl.BlockSpec((AE_H, KB), lambda k: (0, k)),
                  _full((1, AE_H))],
        out_specs=pl.BlockSpec((1, AE_H), lambda k: (0, 0)),
        out_shape=jax.ShapeDtypeStruct((1, AE_H), jnp.float32),
    )(g, w, b)


def _tail(e, p):
    args = (e, p['enc_W1'], p['enc_b1'][None, :], p['enc_W2'],
            p['enc_b2'][None, :], p['dec_W0'], p['dec_b0'][None, :],
            p['dec_W1'], p['dec_b1'][None, :], p['dec_W2'],
            p['dec_b2'][None, :])
    return pl.pallas_call(
        _tail_body,
        in_specs=[_full(a.shape) for a in args],
        out_specs=_full((1, NUM_GENES)),
        out_shape=jax.ShapeDtypeStruct((1, NUM_GENES), jnp.float32),
    )(*args)


EB = 128
NSUB = 16
NWORK = 2 * NSUB
BLK_W = 80
NBLKP = NWORK * BLK_W
EPAD = NBLKP * EB - E
NCHUNK = N // EB
NREM = N - NCHUNK * EB
NBUF = 2
HALF = 40
NPH = BLK_W // HALF


@functools.cache
def _segsum_sc_build():
    mesh = plsc.VectorSubcoreMesh(core_axis_name="c", subcore_axis_name="s")
    return functools.partial(
        pl.kernel, mesh=mesh,
        out_type=jax.ShapeDtypeStruct((2, N, H), jnp.float32),
        scratch_types=[
            pltpu.VMEM_SHARED((N + EB, H), jnp.float32),
            pltpu.VMEM((HALF, EB), jnp.int32),
            pltpu.VMEM((HALF, EB), jnp.int32),
            pltpu.VMEM((NBUF, EB, H), jnp.float32),
            pltpu.SemaphoreType.DMA,
            pltpu.SemaphoreType.DMA,
        ],
    )(_segsum_sc_body)


def _segsum_sc_body(h_hbm, src_hbm, dst_hbm, zeros_hbm, out_hbm,
                    acc, src_v, dst_v, rows4, *sems):
    c = lax.axis_index("c")
    s = lax.axis_index("s")
    wid = c * NSUB + s
    rows_v = rows4.at[0]

    pltpu.sync_copy(zeros_hbm, rows_v)
    for tt in range(5):
        t = s + NSUB * tt
        off = pl.multiple_of(t * EB, EB)
        if tt < 4:
            pltpu.sync_copy(rows_v, acc.at[pl.ds(off, EB)])
        else:
            @pl.when(t < NCHUNK)
            def _():
                pltpu.sync_copy(rows_v, acc.at[pl.ds(off, EB)])

    @pl.when(s == NSUB - 1)
    def _():
        pltpu.sync_copy(rows_v.at[pl.ds(0, NREM)],
                        acc.at[pl.ds(NCHUNK * EB, NREM)])
        pltpu.sync_copy(rows_v, acc.at[pl.ds(N, EB)])
    plsc.subcore_barrier()

    for ph in range(NPH):
        pb = pl.multiple_of(wid * BLK_W + ph * HALF, 8)
        pltpu.sync_copy(src_hbm.at[pl.ds(pb, HALF)], src_v)
        pltpu.sync_copy(dst_hbm.at[pl.ds(pb, HALF)], dst_v)
        for b in range(NBUF):
            pltpu.async_copy(h_hbm.at[src_v.at[b]], rows4.at[b], sems[b])

        def body(g, carry):
            for b in range(NBUF):
                blk = g * NBUF + b
                pltpu.make_async_copy(h_hbm.at[src_v.at[blk]], rows4.at[b],
                                      sems[b]).wait()
                pltpu.sync_copy(rows4.at[b], acc.at[dst_v.at[blk]], add=True)
                nxt = blk + NBUF

                @pl.when(nxt < HALF)
                def _():
                    pltpu.async_copy(h_hbm.at[src_v.at[nxt]], rows4.at[b],
                                     sems[b])
            return carry

        lax.fori_loop(0, HALF // NBUF, body, 0)
    plsc.subcore_barrier()

    for tt in range(5):
        t = s + NSUB * tt
        off = pl.multiple_of(t * EB, EB)
        if tt < 4:
            pltpu.sync_copy(acc.at[pl.ds(off, EB)], rows_v)
            pltpu.sync_copy(rows_v, out_hbm.at[c, pl.ds(off, EB)])
        else:
            @pl.when(t < NCHUNK)
            def _():
                pltpu.sync_copy(acc.at[pl.ds(off, EB)], rows_v)
                pltpu.sync_copy(rows_v, out_hbm.at[c, pl.ds(off, EB)])

    @pl.when(s == NSUB - 1)
    def _():
        pltpu.sync_copy(acc.at[pl.ds(NCHUNK * EB, NREM)],
                        rows_v.at[pl.ds(0, NREM)])
        pltpu.sync_copy(rows_v.at[pl.ds(0, NREM)],
                        out_hbm.at[c, pl.ds(NCHUNK * EB, NREM)])


def _segsum(h, src2d, dst2d, zeros):
    return _segsum_sc_build()(h, src2d, dst2d, zeros)


def kernel(x, edge_index, params):
    p = params
    pad_src = jnp.zeros((EPAD,), jnp.int32)
    pad_dst = jnp.full((EPAD,), N, jnp.int32)
    src2d = jnp.concatenate([edge_index[0], pad_src]).reshape(NBLKP, EB)
    dst2d = jnp.concatenate([edge_index[1], pad_dst]).reshape(NBLKP, EB)
    zeros = jnp.zeros((EB, H), jnp.float32)
    h = _pre(x, p['W_pre'], p['b_pre'][None, :])
    for i in range(3):
        msg = _segsum(h, src2d, dst2d, zeros)
        h, cs, cs2 = _gin(h, msg, p['c%d_W1' % i], p['c%d_b1' % i][None, :],
                          p['c%d_W2' % i], p['c%d_b2' % i][None, :])
        if i < 2:
            h = _bn(h, cs, cs2, p['bn%d_g' % i][None, :],
                    p['bn%d_b' % i][None, :])
    h8 = _post(h, p['post_W1'], p['post_b1'][None, :],
               p['post_W2'], p['post_b2'][None, :])
    g = h8.reshape(1, N * OUT)
    e0 = _enc0(g, p['enc_W0'], p['enc_b0'][None, :])
    return _tail(e0, p)
